# baseline (speedup 1.0000x reference)
import functools

import jax
import jax.numpy as jnp
from jax import lax
from jax.experimental import pallas as pl
from jax.experimental.pallas import tpu as pltpu
from jax.experimental.pallas import tpu_sc as plsc

NUM_ITEMS = 1000000
LATENT_DIM = 16
BATCH = 16384
HIST = 50

NC = 2
NS = 16
NW = NC * NS

TOTAL = BATCH * HIST
PER_W = TOTAL // NW
G = 128
CHUNK = 1024
GROUPS_PER_CHUNK = CHUNK // G
CHUNKS_PER_W = PER_W // CHUNK
GROUPS_PER_W = PER_W // G

BT = 4096
BB = 256


def _t_small(x):
  eye = jnp.eye(LATENT_DIM, dtype=jnp.float32)
  return jax.lax.dot_general(
      x, eye, (((0,), (0,)), ((), ())), preferred_element_type=jnp.float32)


def _t_big(x):
  eye = jnp.eye(BB, dtype=jnp.float32)
  return jax.lax.dot_general(
      x, eye, (((0,), (0,)), ((), ())), preferred_element_type=jnp.float32)


def _tin_body(tp_ref, tc_ref, op_ref, oc_ref):
  op_ref[...] = _t_small(tp_ref[...])
  oc_ref[...] = _t_small(tc_ref[...])


def _tc_transpose_in(table_p_t, table_c_t):
  grid = (pl.cdiv(NUM_ITEMS, BT),)
  return pl.pallas_call(
      _tin_body,
      grid=grid,
      in_specs=[
          pl.BlockSpec((LATENT_DIM, BT), lambda i: (0, i)),
          pl.BlockSpec((LATENT_DIM, BT), lambda i: (0, i)),
      ],
      out_specs=[
          pl.BlockSpec((BT, LATENT_DIM), lambda i: (i, 0)),
          pl.BlockSpec((BT, LATENT_DIM), lambda i: (i, 0)),
      ],
      out_shape=[
          jax.ShapeDtypeStruct((NUM_ITEMS, LATENT_DIM), jnp.float32),
          jax.ShapeDtypeStruct((NUM_ITEMS, LATENT_DIM), jnp.float32),
      ],
  )(table_p_t, table_c_t)


def _sc_gather(idx2d, table_p, table_c):
  mesh = plsc.VectorSubcoreMesh(core_axis_name="c", subcore_axis_name="s")

  @functools.partial(
      pl.kernel,
      out_type=(
          jax.ShapeDtypeStruct((TOTAL, LATENT_DIM), jnp.float32),
          jax.ShapeDtypeStruct((TOTAL, LATENT_DIM), jnp.float32),
      ),
      mesh=mesh,
      compiler_params=pltpu.CompilerParams(use_tc_tiling_on_sc=False),
      scratch_types=[
          pltpu.VMEM((GROUPS_PER_CHUNK, G), jnp.int32),
          pltpu.VMEM((CHUNK, LATENT_DIM), jnp.float32),
          pltpu.VMEM((CHUNK, LATENT_DIM), jnp.float32),
          pltpu.SemaphoreType.DMA,
          pltpu.SemaphoreType.DMA,
      ],
  )
  def k(idx_hbm, tp_hbm, tc_hbm, outp_hbm, outc_hbm, idx_v, bufp, bufc,
        semp, semc):
    wid = lax.axis_index("s") * NC + lax.axis_index("c")
    wgbase = wid * GROUPS_PER_W
    wrbase = wid * PER_W

    def body(kk, carry):
      gbase = wgbase + kk * GROUPS_PER_CHUNK
      rbase = wrbase + kk * CHUNK
      pltpu.sync_copy(idx_hbm.at[pl.ds(gbase, GROUPS_PER_CHUNK)], idx_v)
      waits = []
      for j in range(GROUPS_PER_CHUNK):
        waits.append(
            pltpu.async_copy(tp_hbm.at[idx_v.at[j]],
                             bufp.at[pl.ds(j * G, G)], semp))
        waits.append(
            pltpu.async_copy(tc_hbm.at[idx_v.at[j]],
                             bufc.at[pl.ds(j * G, G)], semc))
      for w in waits:
        w.wait()
      pltpu.sync_copy(bufp, outp_hbm.at[pl.ds(rbase, CHUNK)])
      pltpu.sync_copy(bufc, outc_hbm.at[pl.ds(rbase, CHUNK)])
      return carry

    lax.fori_loop(0, CHUNKS_PER_W, body, 0)

  return k(idx2d, table_p, table_c)


def _tout_body(gp_ref, gc_ref, w_ref, b_ref, op_ref, oc_ref, r_ref):
  pt = _t_big(gp_ref[...])
  ct = _t_big(gc_ref[...])
  op_ref[...] = pt
  oc_ref[...] = ct
  pred = jax.lax.dot_general(
      w_ref[...], pt + ct, (((1,), (0,)), ((), ())),
      preferred_element_type=jnp.float32)
  r_ref[...] = jax.nn.sigmoid(pred + b_ref[...]).reshape(1, 1, BB)


def _tc_transpose_out(gp_h, gc_h, w_t, b11):
  jb = BATCH // BB
  grid = (HIST, jb)
  return pl.pallas_call(
      _tout_body,
      grid=grid,
      in_specs=[
          pl.BlockSpec((BB, LATENT_DIM), lambda h, j: (h * jb + j, 0)),
          pl.BlockSpec((BB, LATENT_DIM), lambda h, j: (h * jb + j, 0)),
          pl.BlockSpec((1, LATENT_DIM), lambda h, j: (0, 0)),
          pl.BlockSpec((1, 1), lambda h, j: (0, 0)),
      ],
      out_specs=[
          pl.BlockSpec((LATENT_DIM, BB), lambda h, j: (h, j)),
          pl.BlockSpec((LATENT_DIM, BB), lambda h, j: (h, j)),
          pl.BlockSpec((1, 1, BB), lambda h, j: (h, 0, j)),
      ],
      out_shape=[
          jax.ShapeDtypeStruct((HIST * LATENT_DIM, BATCH), jnp.float32),
          jax.ShapeDtypeStruct((HIST * LATENT_DIM, BATCH), jnp.float32),
          jax.ShapeDtypeStruct((HIST, 1, BATCH), jnp.float32),
      ],
  )(gp_h, gc_h, w_t, b11)


def kernel(item_indices, item_personality_table, item_commonality_table,
           affine_W, affine_b):
  idx2d = item_indices.astype(jnp.int32).T.reshape(TOTAL // G, G)
  tp_lin, tc_lin = _tc_transpose_in(
      item_personality_table.T, item_commonality_table.T)
  gp_h, gc_h = _sc_gather(idx2d, tp_lin, tc_lin)
  gp_t, gc_t, rating_h = _tc_transpose_out(
      gp_h, gc_h, affine_W.T, affine_b.reshape(1, 1))
  rating = rating_h.transpose(2, 0, 1)
  gp = gp_t.reshape(HIST, LATENT_DIM, BATCH).transpose(2, 0, 1)
  gc = gc_t.reshape(HIST, LATENT_DIM, BATCH).transpose(2, 0, 1)
  return (rating, gp, gc)


# speedup vs baseline: 2.3227x; 2.3227x over previous
import functools

import jax
import jax.numpy as jnp
from jax import lax
from jax.experimental import pallas as pl
from jax.experimental.pallas import tpu as pltpu
from jax.experimental.pallas import tpu_sc as plsc

NUM_ITEMS = 1000000
LATENT_DIM = 16
BATCH = 16384
HIST = 50

NC = 2
NS = 16
NW = NC * NS

TOTAL = BATCH * HIST
PER_W = TOTAL // NW
G = 128
CHUNK = 1024
GROUPS_PER_CHUNK = CHUNK // G
CHUNKS_PER_W = PER_W // CHUNK
GROUPS_PER_W = PER_W // G

BT = 16384
PACK = 128 // LATENT_DIM


def _tin_body(tp_ref, tc_ref, op_ref, oc_ref):
  op_ref[...] = jnp.swapaxes(tp_ref[...], 0, 1)
  oc_ref[...] = jnp.swapaxes(tc_ref[...], 0, 1)


def _tc_transpose_in(table_p_t, table_c_t):
  grid = (pl.cdiv(NUM_ITEMS, BT),)
  return pl.pallas_call(
      _tin_body,
      grid=grid,
      in_specs=[
          pl.BlockSpec((LATENT_DIM, BT), lambda i: (0, i)),
          pl.BlockSpec((LATENT_DIM, BT), lambda i: (0, i)),
      ],
      out_specs=[
          pl.BlockSpec((BT, LATENT_DIM), lambda i: (i, 0)),
          pl.BlockSpec((BT, LATENT_DIM), lambda i: (i, 0)),
      ],
      out_shape=[
          jax.ShapeDtypeStruct((NUM_ITEMS, LATENT_DIM), jnp.float32),
          jax.ShapeDtypeStruct((NUM_ITEMS, LATENT_DIM), jnp.float32),
      ],
  )(table_p_t, table_c_t)


_AR16 = None


def _sc_gather(idx2d, table_p, table_c):
  mesh = plsc.VectorSubcoreMesh(core_axis_name="c", subcore_axis_name="s")

  @functools.partial(
      pl.kernel,
      out_type=(
          jax.ShapeDtypeStruct((TOTAL * LATENT_DIM,), jnp.float32),
          jax.ShapeDtypeStruct((TOTAL * LATENT_DIM,), jnp.float32),
      ),
      mesh=mesh,
      compiler_params=pltpu.CompilerParams(
          use_tc_tiling_on_sc=False, needs_layout_passes=False),
      scratch_types=[
          pltpu.VMEM((GROUPS_PER_CHUNK, G), jnp.int32),
          pltpu.VMEM((CHUNK, LATENT_DIM), jnp.float32),
          pltpu.VMEM((CHUNK, LATENT_DIM), jnp.float32),
          pltpu.VMEM((CHUNK * LATENT_DIM,), jnp.float32),
          pltpu.VMEM((CHUNK * LATENT_DIM,), jnp.float32),
          pltpu.SemaphoreType.DMA,
          pltpu.SemaphoreType.DMA,
      ],
  )
  def k(idx_hbm, tp_hbm, tc_hbm, outp_hbm, outc_hbm, idx_v, bufp, bufc,
        bufpt, bufct, semp, semc):
    wid = lax.axis_index("s") * NC + lax.axis_index("c")
    wgbase = wid * GROUPS_PER_W
    wrbase = wid * PER_W
    ar16 = jnp.arange(LATENT_DIM, dtype=jnp.int32)

    def body(kk, carry):
      gbase = wgbase + kk * GROUPS_PER_CHUNK
      rbase = wrbase + kk * CHUNK
      pltpu.sync_copy(idx_hbm.at[pl.ds(gbase, GROUPS_PER_CHUNK)], idx_v)
      waits = []
      for j in range(GROUPS_PER_CHUNK):
        waits.append(
            pltpu.async_copy(tp_hbm.at[idx_v.at[j]],
                             bufp.at[pl.ds(j * G, G)], semp))
        waits.append(
            pltpu.async_copy(tc_hbm.at[idx_v.at[j]],
                             bufc.at[pl.ds(j * G, G)], semc))
      for w in waits:
        w.wait()

      def tbody(t, carry2):
        rows0 = (t // LATENT_DIM) * G
        d = t % LATENT_DIM
        dcol = jnp.full((LATENT_DIM,), d, dtype=jnp.int32)
        base_out = t * G
        for j0 in range(G // LATENT_DIM):
          rsel = rows0 + j0 * LATENT_DIM + ar16
          vp = plsc.load_gather(bufp, [rsel, dcol])
          vc = plsc.load_gather(bufc, [rsel, dcol])
          bufpt[pl.ds(base_out + j0 * LATENT_DIM, LATENT_DIM)] = vp
          bufct[pl.ds(base_out + j0 * LATENT_DIM, LATENT_DIM)] = vc
        return carry2

      lax.fori_loop(0, GROUPS_PER_CHUNK * LATENT_DIM, tbody, 0)

      pltpu.sync_copy(bufpt,
                      outp_hbm.at[pl.ds(rbase * LATENT_DIM,
                                        CHUNK * LATENT_DIM)])
      pltpu.sync_copy(bufct,
                      outc_hbm.at[pl.ds(rbase * LATENT_DIM,
                                        CHUNK * LATENT_DIM)])
      return carry

    lax.fori_loop(0, CHUNKS_PER_W, body, 0)

  return k(idx2d, table_p, table_c)


GB_PER_H = BATCH // G


def _tout_body(gp_ref, gc_ref, w_ref, b_ref, op_ref, oc_ref, r_ref):
  p3 = gp_ref[...].reshape(GB_PER_H, LATENT_DIM, G)
  c3 = gc_ref[...].reshape(GB_PER_H, LATENT_DIM, G)
  pt = jnp.transpose(p3, (1, 0, 2)).reshape(LATENT_DIM, BATCH)
  ct = jnp.transpose(c3, (1, 0, 2)).reshape(LATENT_DIM, BATCH)
  op_ref[...] = pt
  oc_ref[...] = ct
  s3 = p3 + c3
  acc = jnp.zeros((GB_PER_H, G), dtype=jnp.float32)
  w = w_ref[...]
  for d in range(LATENT_DIM):
    acc = acc + s3[:, d, :] * w[0, d]
  r_ref[...] = jax.nn.sigmoid(acc + b_ref[0, 0]).reshape(1, GB_PER_H, G)


def _tc_transpose_out(gp2, gc2, w_t, b11):
  grid = (HIST,)
  return pl.pallas_call(
      _tout_body,
      grid=grid,
      in_specs=[
          pl.BlockSpec((GB_PER_H, LATENT_DIM * G), lambda h: (h, 0)),
          pl.BlockSpec((GB_PER_H, LATENT_DIM * G), lambda h: (h, 0)),
          pl.BlockSpec((1, LATENT_DIM), lambda h: (0, 0)),
          pl.BlockSpec((1, 1), lambda h: (0, 0)),
      ],
      out_specs=[
          pl.BlockSpec((LATENT_DIM, BATCH), lambda h: (h, 0)),
          pl.BlockSpec((LATENT_DIM, BATCH), lambda h: (h, 0)),
          pl.BlockSpec((1, GB_PER_H, G), lambda h: (h, 0, 0)),
      ],
      out_shape=[
          jax.ShapeDtypeStruct((HIST * LATENT_DIM, BATCH), jnp.float32),
          jax.ShapeDtypeStruct((HIST * LATENT_DIM, BATCH), jnp.float32),
          jax.ShapeDtypeStruct((HIST, GB_PER_H, G), jnp.float32),
      ],
  )(gp2, gc2, w_t, b11)


def kernel(item_indices, item_personality_table, item_commonality_table,
           affine_W, affine_b):
  idx2d = item_indices.astype(jnp.int32).T.reshape(TOTAL // G, G)
  tp_pk, tc_pk = _tc_transpose_in(
      item_personality_table.T, item_commonality_table.T)
  gp_f, gc_f = _sc_gather(idx2d, tp_pk, tc_pk)
  gp_t, gc_t, rating_h = _tc_transpose_out(
      gp_f.reshape(TOTAL // G, LATENT_DIM * G),
      gc_f.reshape(TOTAL // G, LATENT_DIM * G),
      affine_W.T, affine_b.reshape(1, 1))
  rating = rating_h.reshape(HIST, 1, BATCH).transpose(2, 0, 1)
  gp = gp_t.reshape(HIST, LATENT_DIM, BATCH).transpose(2, 0, 1)
  gc = gc_t.reshape(HIST, LATENT_DIM, BATCH).transpose(2, 0, 1)
  return (rating, gp, gc)


# speedup vs baseline: 2.4804x; 1.0679x over previous
import functools

import jax
import jax.numpy as jnp
from jax import lax
from jax.experimental import pallas as pl
from jax.experimental.pallas import tpu as pltpu
from jax.experimental.pallas import tpu_sc as plsc

NUM_ITEMS = 1000000
LATENT_DIM = 16
BATCH = 16384
HIST = 50

NC = 2
NS = 16
NW = NC * NS

TOTAL = BATCH * HIST
PER_W = TOTAL // NW
G = 128
CHUNK = 1024
GROUPS_PER_CHUNK = CHUNK // G
CHUNKS_PER_W = PER_W // CHUNK
GROUPS_PER_W = PER_W // G

BT = 16384
GB_PER_H = BATCH // G


def _tin_body(t_ref, o_ref):
  o_ref[...] = jnp.swapaxes(t_ref[...], 0, 1)


def _tc_transpose_in(table_t):
  grid = (pl.cdiv(NUM_ITEMS, BT),)
  return pl.pallas_call(
      _tin_body,
      grid=grid,
      in_specs=[pl.BlockSpec((LATENT_DIM, BT), lambda i: (0, i))],
      out_specs=pl.BlockSpec((BT, LATENT_DIM), lambda i: (i, 0)),
      out_shape=jax.ShapeDtypeStruct((NUM_ITEMS, LATENT_DIM), jnp.float32),
  )(table_t)


def _sc_gather(idx2d, table):
  mesh = plsc.VectorSubcoreMesh(core_axis_name="c", subcore_axis_name="s")

  @functools.partial(
      pl.kernel,
      out_type=jax.ShapeDtypeStruct((TOTAL * LATENT_DIM,), jnp.float32),
      mesh=mesh,
      compiler_params=pltpu.CompilerParams(
          use_tc_tiling_on_sc=False, needs_layout_passes=False),
      scratch_types=[
          pltpu.VMEM((GROUPS_PER_CHUNK, G), jnp.int32),
          pltpu.VMEM((CHUNK, LATENT_DIM), jnp.float32),
          pltpu.VMEM((CHUNK * LATENT_DIM,), jnp.float32),
          pltpu.SemaphoreType.DMA,
      ],
  )
  def k(idx_hbm, t_hbm, out_hbm, idx_v, buf, buft, sem):
    wid = lax.axis_index("s") * NC + lax.axis_index("c")
    wgbase = wid * GROUPS_PER_W
    wrbase = wid * PER_W
    ar16 = jnp.arange(LATENT_DIM, dtype=jnp.int32)

    def body(kk, carry):
      gbase = wgbase + kk * GROUPS_PER_CHUNK
      rbase = wrbase + kk * CHUNK
      pltpu.sync_copy(idx_hbm.at[pl.ds(gbase, GROUPS_PER_CHUNK)], idx_v)
      waits = []
      for j in range(GROUPS_PER_CHUNK):
        waits.append(
            pltpu.async_copy(t_hbm.at[idx_v.at[j]],
                             buf.at[pl.ds(j * G, G)], sem))
      for w in waits:
        w.wait()

      def tbody(g, carry2):
        rows0 = g * G
        base_g = g * G * LATENT_DIM
        rsels = [rows0 + j0 * LATENT_DIM + ar16
                 for j0 in range(G // LATENT_DIM)]
        for d in range(LATENT_DIM):
          dcol = jnp.full((LATENT_DIM,), d, dtype=jnp.int32)
          base_out = base_g + d * G
          for j0 in range(G // LATENT_DIM):
            v = plsc.load_gather(buf, [rsels[j0], dcol])
            buft[pl.ds(base_out + j0 * LATENT_DIM, LATENT_DIM)] = v
        return carry2

      lax.fori_loop(0, GROUPS_PER_CHUNK, tbody, 0)

      pltpu.sync_copy(buft,
                      out_hbm.at[pl.ds(rbase * LATENT_DIM,
                                       CHUNK * LATENT_DIM)])
      return carry

    lax.fori_loop(0, CHUNKS_PER_W, body, 0)

  return k(idx2d, table)


def _toutp_body(gp_ref, op_ref):
  p3 = gp_ref[...].reshape(GB_PER_H, LATENT_DIM, G)
  op_ref[...] = jnp.transpose(p3, (1, 0, 2)).reshape(LATENT_DIM, BATCH)


def _tc_transpose_out_p(gp2):
  grid = (HIST,)
  return pl.pallas_call(
      _toutp_body,
      grid=grid,
      in_specs=[pl.BlockSpec((GB_PER_H, LATENT_DIM * G), lambda h: (h, 0))],
      out_specs=pl.BlockSpec((LATENT_DIM, BATCH), lambda h: (h, 0)),
      out_shape=jax.ShapeDtypeStruct((HIST * LATENT_DIM, BATCH),
                                     jnp.float32),
  )(gp2)


def _toutc_body(gc_ref, gp_ref, w_ref, b_ref, oc_ref, r_ref):
  c3 = gc_ref[...].reshape(GB_PER_H, LATENT_DIM, G)
  p3 = gp_ref[...].reshape(GB_PER_H, LATENT_DIM, G)
  oc_ref[...] = jnp.transpose(c3, (1, 0, 2)).reshape(LATENT_DIM, BATCH)
  s3 = p3 + c3
  acc = jnp.zeros((GB_PER_H, G), dtype=jnp.float32)
  w = w_ref[...]
  for d in range(LATENT_DIM):
    acc = acc + s3[:, d, :] * w[0, d]
  r_ref[...] = jax.nn.sigmoid(acc + b_ref[0, 0]).reshape(1, GB_PER_H, G)


def _tc_transpose_out_c(gc2, gp2, w_t, b11):
  grid = (HIST,)
  return pl.pallas_call(
      _toutc_body,
      grid=grid,
      in_specs=[
          pl.BlockSpec((GB_PER_H, LATENT_DIM * G), lambda h: (h, 0)),
          pl.BlockSpec((GB_PER_H, LATENT_DIM * G), lambda h: (h, 0)),
          pl.BlockSpec((1, LATENT_DIM), lambda h: (0, 0)),
          pl.BlockSpec((1, 1), lambda h: (0, 0)),
      ],
      out_specs=[
          pl.BlockSpec((LATENT_DIM, BATCH), lambda h: (h, 0)),
          pl.BlockSpec((1, GB_PER_H, G), lambda h: (h, 0, 0)),
      ],
      out_shape=[
          jax.ShapeDtypeStruct((HIST * LATENT_DIM, BATCH), jnp.float32),
          jax.ShapeDtypeStruct((HIST, GB_PER_H, G), jnp.float32),
      ],
  )(gc2, gp2, w_t, b11)


def kernel(item_indices, item_personality_table, item_commonality_table,
           affine_W, affine_b):
  idx2d = item_indices.astype(jnp.int32).T.reshape(TOTAL // G, G)
  tp_lin = _tc_transpose_in(item_personality_table.T)
  gp_f = _sc_gather(idx2d, tp_lin)
  tc_lin = _tc_transpose_in(item_commonality_table.T)
  gc_f = _sc_gather(idx2d, tc_lin)
  gp2 = gp_f.reshape(TOTAL // G, LATENT_DIM * G)
  gc2 = gc_f.reshape(TOTAL // G, LATENT_DIM * G)
  gp_t = _tc_transpose_out_p(gp2)
  gc_t, rating_h = _tc_transpose_out_c(
      gc2, gp2, affine_W.T, affine_b.reshape(1, 1))
  rating = rating_h.reshape(HIST, 1, BATCH).transpose(2, 0, 1)
  gp = gp_t.reshape(HIST, LATENT_DIM, BATCH).transpose(2, 0, 1)
  gc = gc_t.reshape(HIST, LATENT_DIM, BATCH).transpose(2, 0, 1)
  return (rating, gp, gc)


# speedup vs baseline: 2.9133x; 1.1745x over previous
import functools

import jax
import jax.numpy as jnp
from jax import lax
from jax.experimental import pallas as pl
from jax.experimental.pallas import tpu as pltpu
from jax.experimental.pallas import tpu_sc as plsc

NUM_ITEMS = 1000000
LATENT_DIM = 16
BATCH = 16384
HIST = 50

NC = 2
NS = 16
NW = NC * NS

TOTAL = BATCH * HIST
PER_W = TOTAL // NW
G = 128
CHUNK = 1024
GROUPS_PER_CHUNK = CHUNK // G
CHUNKS_PER_W = PER_W // CHUNK
GROUPS_PER_W = PER_W // G

BT = 16384
GB_PER_H = BATCH // G


def _tin_body(t_ref, o_ref):
  o_ref[...] = jnp.swapaxes(t_ref[...], 0, 1)


def _tc_transpose_in(table_t):
  grid = (pl.cdiv(NUM_ITEMS, BT),)
  return pl.pallas_call(
      _tin_body,
      grid=grid,
      in_specs=[pl.BlockSpec((LATENT_DIM, BT), lambda i: (0, i))],
      out_specs=pl.BlockSpec((BT, LATENT_DIM), lambda i: (i, 0)),
      out_shape=jax.ShapeDtypeStruct((NUM_ITEMS, LATENT_DIM), jnp.float32),
  )(table_t)


def _sc_gather(idx2d, table):
  mesh = plsc.VectorSubcoreMesh(core_axis_name="c", subcore_axis_name="s")

  @functools.partial(
      pl.kernel,
      out_type=jax.ShapeDtypeStruct((TOTAL * LATENT_DIM,), jnp.float32),
      mesh=mesh,
      compiler_params=pltpu.CompilerParams(
          use_tc_tiling_on_sc=False, needs_layout_passes=False),
      scratch_types=[
          pltpu.VMEM((GROUPS_PER_CHUNK, G), jnp.int32),
          pltpu.VMEM((CHUNK, LATENT_DIM), jnp.float32),
          pltpu.VMEM((CHUNK * LATENT_DIM,), jnp.float32),
          pltpu.SemaphoreType.DMA,
      ],
  )
  def k(idx_hbm, t_hbm, out_hbm, idx_v, buf, buft, sem):
    wid = lax.axis_index("s") * NC + lax.axis_index("c")
    wgbase = wid * GROUPS_PER_W
    wrbase = wid * PER_W
    ar16 = jnp.arange(LATENT_DIM, dtype=jnp.int32)

    def body(kk, carry):
      gbase = wgbase + kk * GROUPS_PER_CHUNK
      rbase = wrbase + kk * CHUNK
      pltpu.sync_copy(idx_hbm.at[pl.ds(gbase, GROUPS_PER_CHUNK)], idx_v)
      waits = []
      for j in range(GROUPS_PER_CHUNK):
        waits.append(
            pltpu.async_copy(t_hbm.at[idx_v.at[j]],
                             buf.at[pl.ds(j * G, G)], sem))
      for w in waits:
        w.wait()

      def tbody(g, carry2):
        rows0 = g * G
        base_g = g * G * LATENT_DIM
        rsels = [rows0 + j0 * LATENT_DIM + ar16
                 for j0 in range(G // LATENT_DIM)]
        for d in range(LATENT_DIM):
          dcol = jnp.full((LATENT_DIM,), d, dtype=jnp.int32)
          base_out = base_g + d * G
          for j0 in range(G // LATENT_DIM):
            v = plsc.load_gather(buf, [rsels[j0], dcol])
            buft[pl.ds(base_out + j0 * LATENT_DIM, LATENT_DIM)] = v
        return carry2

      lax.fori_loop(0, GROUPS_PER_CHUNK, tbody, 0)

      pltpu.sync_copy(buft,
                      out_hbm.at[pl.ds(rbase * LATENT_DIM,
                                       CHUNK * LATENT_DIM)])
      return carry

    lax.fori_loop(0, CHUNKS_PER_W, body, 0)

  return k(idx2d, table)


def _toutp_body(gp_ref, op_ref):
  p3 = gp_ref[...].reshape(GB_PER_H, LATENT_DIM, G)
  op_ref[...] = jnp.transpose(p3, (1, 0, 2)).reshape(LATENT_DIM, BATCH)


def _tc_transpose_out_p(gp2):
  grid = (HIST,)
  return pl.pallas_call(
      _toutp_body,
      grid=grid,
      in_specs=[pl.BlockSpec((GB_PER_H, LATENT_DIM * G), lambda h: (h, 0))],
      out_specs=pl.BlockSpec((LATENT_DIM, BATCH), lambda h: (h, 0)),
      out_shape=jax.ShapeDtypeStruct((HIST * LATENT_DIM, BATCH),
                                     jnp.float32),
  )(gp2)


def _toutc_body(gc_ref, gp_ref, w_ref, b_ref, oc_ref, r_ref):
  c3 = gc_ref[...].reshape(GB_PER_H, LATENT_DIM, G)
  p3 = gp_ref[...].reshape(GB_PER_H, LATENT_DIM, G)
  oc_ref[...] = jnp.transpose(c3, (1, 0, 2)).reshape(LATENT_DIM, BATCH)
  s3 = p3 + c3
  acc = jnp.zeros((GB_PER_H, G), dtype=jnp.float32)
  w = w_ref[...]
  for d in range(LATENT_DIM):
    acc = acc + s3[:, d, :] * w[0, d]
  r_ref[...] = jax.nn.sigmoid(acc + b_ref[0, 0]).reshape(1, GB_PER_H, G)


def _tc_transpose_out_c(gc2, gp2, w_t, b11):
  grid = (HIST,)
  return pl.pallas_call(
      _toutc_body,
      grid=grid,
      in_specs=[
          pl.BlockSpec((GB_PER_H, LATENT_DIM * G), lambda h: (h, 0)),
          pl.BlockSpec((GB_PER_H, LATENT_DIM * G), lambda h: (h, 0)),
          pl.BlockSpec((1, LATENT_DIM), lambda h: (0, 0)),
          pl.BlockSpec((1, 1), lambda h: (0, 0)),
      ],
      out_specs=[
          pl.BlockSpec((LATENT_DIM, BATCH), lambda h: (h, 0)),
          pl.BlockSpec((1, GB_PER_H, G), lambda h: (h, 0, 0)),
      ],
      out_shape=[
          jax.ShapeDtypeStruct((HIST * LATENT_DIM, BATCH), jnp.float32),
          jax.ShapeDtypeStruct((HIST, GB_PER_H, G), jnp.float32),
      ],
  )(gc2, gp2, w_t, b11)


def kernel(item_indices, item_personality_table, item_commonality_table,
           affine_W, affine_b):
  idx2d = item_indices.astype(jnp.int32).T.reshape(TOTAL // G, G)
  gp_f = _sc_gather(idx2d, item_personality_table)
  gc_f = _sc_gather(idx2d, item_commonality_table)
  gp2 = gp_f.reshape(TOTAL // G, LATENT_DIM * G)
  gc2 = gc_f.reshape(TOTAL // G, LATENT_DIM * G)
  gp_t = _tc_transpose_out_p(gp2)
  gc_t, rating_h = _tc_transpose_out_c(
      gc2, gp2, affine_W.T, affine_b.reshape(1, 1))
  rating = rating_h.reshape(HIST, 1, BATCH).transpose(2, 0, 1)
  gp = gp_t.reshape(HIST, LATENT_DIM, BATCH).transpose(2, 0, 1)
  gc = gc_t.reshape(HIST, LATENT_DIM, BATCH).transpose(2, 0, 1)
  return (rating, gp, gc)


# speedup vs baseline: 3.0107x; 1.0334x over previous
import functools

import jax
import jax.numpy as jnp
from jax import lax
from jax.experimental import pallas as pl
from jax.experimental.pallas import tpu as pltpu
from jax.experimental.pallas import tpu_sc as plsc

NUM_ITEMS = 1000000
LATENT_DIM = 16
BATCH = 16384
HIST = 50

NC = 2
NS = 16
NW = NC * NS

TOTAL = BATCH * HIST
PER_W = TOTAL // NW
G = 128
CHUNK = 1024
GROUPS_PER_CHUNK = CHUNK // G
CHUNKS_PER_W = PER_W // CHUNK
GROUPS_PER_W = PER_W // G

BT = 16384
GB_PER_H = BATCH // G


def _tin_body(t_ref, o_ref):
  o_ref[...] = jnp.swapaxes(t_ref[...], 0, 1)


def _tc_transpose_in(table_t):
  grid = (pl.cdiv(NUM_ITEMS, BT),)
  return pl.pallas_call(
      _tin_body,
      grid=grid,
      in_specs=[pl.BlockSpec((LATENT_DIM, BT), lambda i: (0, i))],
      out_specs=pl.BlockSpec((BT, LATENT_DIM), lambda i: (i, 0)),
      out_shape=jax.ShapeDtypeStruct((NUM_ITEMS, LATENT_DIM), jnp.float32),
  )(table_t)


def _sc_gather(idx2d, table):
  mesh = plsc.VectorSubcoreMesh(core_axis_name="c", subcore_axis_name="s")

  @functools.partial(
      pl.kernel,
      out_type=jax.ShapeDtypeStruct((TOTAL * LATENT_DIM,), jnp.float32),
      mesh=mesh,
      compiler_params=pltpu.CompilerParams(
          use_tc_tiling_on_sc=False, needs_layout_passes=False),
      scratch_types=[
          pltpu.VMEM((GROUPS_PER_CHUNK, G), jnp.int32),
          pltpu.VMEM((CHUNK, LATENT_DIM), jnp.float32),
          pltpu.VMEM((CHUNK * LATENT_DIM,), jnp.float32),
          pltpu.SemaphoreType.DMA,
      ],
  )
  def k(idx_hbm, t_hbm, out_hbm, idx_v, buf, buft, sem):
    wid = lax.axis_index("s") * NC + lax.axis_index("c")
    wgbase = wid * GROUPS_PER_W
    wrbase = wid * PER_W
    ar16 = jnp.arange(LATENT_DIM, dtype=jnp.int32)

    def body(kk, carry):
      gbase = wgbase + kk * GROUPS_PER_CHUNK
      rbase = wrbase + kk * CHUNK
      pltpu.sync_copy(idx_hbm.at[pl.ds(gbase, GROUPS_PER_CHUNK)], idx_v)
      waits = []
      for j in range(GROUPS_PER_CHUNK):
        waits.append(
            pltpu.async_copy(t_hbm.at[idx_v.at[j]],
                             buf.at[pl.ds(j * G, G)], sem))
      for w in waits:
        w.wait()

      def tbody(g, carry2):
        rows0 = g * G
        base_g = g * G * LATENT_DIM
        rsels = [rows0 + j0 * LATENT_DIM + ar16
                 for j0 in range(G // LATENT_DIM)]
        for d in range(LATENT_DIM):
          dcol = jnp.full((LATENT_DIM,), d, dtype=jnp.int32)
          base_out = base_g + d * G
          for j0 in range(G // LATENT_DIM):
            v = plsc.load_gather(buf, [rsels[j0], dcol])
            buft[pl.ds(base_out + j0 * LATENT_DIM, LATENT_DIM)] = v
        return carry2

      lax.fori_loop(0, GROUPS_PER_CHUNK, tbody, 0)

      pltpu.sync_copy(buft,
                      out_hbm.at[pl.ds(rbase * LATENT_DIM,
                                       CHUNK * LATENT_DIM)])
      return carry

    lax.fori_loop(0, CHUNKS_PER_W, body, 0)

  return k(idx2d, table)


def _tout_body(gp_ref, gc_ref, w_ref, b_ref, op_ref, oc_ref, r_ref):
  p3 = gp_ref[...].reshape(GB_PER_H, LATENT_DIM, G)
  c3 = gc_ref[...].reshape(GB_PER_H, LATENT_DIM, G)
  op_ref[...] = jnp.transpose(p3, (1, 0, 2)).reshape(LATENT_DIM, BATCH)
  oc_ref[...] = jnp.transpose(c3, (1, 0, 2)).reshape(LATENT_DIM, BATCH)
  s3 = p3 + c3
  acc = jnp.zeros((GB_PER_H, G), dtype=jnp.float32)
  w = w_ref[...]
  for d in range(LATENT_DIM):
    acc = acc + s3[:, d, :] * w[0, d]
  r_ref[...] = jax.nn.sigmoid(acc + b_ref[0, 0]).reshape(1, GB_PER_H, G)


def _tc_transpose_out(gp2, gc2, w_t, b11):
  grid = (HIST,)
  return pl.pallas_call(
      _tout_body,
      grid=grid,
      in_specs=[
          pl.BlockSpec((GB_PER_H, LATENT_DIM * G), lambda h: (h, 0)),
          pl.BlockSpec((GB_PER_H, LATENT_DIM * G), lambda h: (h, 0)),
          pl.BlockSpec((1, LATENT_DIM), lambda h: (0, 0)),
          pl.BlockSpec((1, 1), lambda h: (0, 0)),
      ],
      out_specs=[
          pl.BlockSpec((LATENT_DIM, BATCH), lambda h: (h, 0)),
          pl.BlockSpec((LATENT_DIM, BATCH), lambda h: (h, 0)),
          pl.BlockSpec((1, GB_PER_H, G), lambda h: (h, 0, 0)),
      ],
      out_shape=[
          jax.ShapeDtypeStruct((HIST * LATENT_DIM, BATCH), jnp.float32),
          jax.ShapeDtypeStruct((HIST * LATENT_DIM, BATCH), jnp.float32),
          jax.ShapeDtypeStruct((HIST, GB_PER_H, G), jnp.float32),
      ],
  )(gp2, gc2, w_t, b11)


def kernel(item_indices, item_personality_table, item_commonality_table,
           affine_W, affine_b):
  idx2d = item_indices.astype(jnp.int32).T.reshape(TOTAL // G, G)
  gp_f = _sc_gather(idx2d, item_personality_table)
  gc_f = _sc_gather(idx2d, item_commonality_table)
  gp2 = gp_f.reshape(TOTAL // G, LATENT_DIM * G)
  gc2 = gc_f.reshape(TOTAL // G, LATENT_DIM * G)
  gp_t, gc_t, rating_h = _tc_transpose_out(
      gp2, gc2, affine_W.T, affine_b.reshape(1, 1))
  rating = rating_h.reshape(HIST, 1, BATCH).transpose(2, 0, 1)
  gp = gp_t.reshape(HIST, LATENT_DIM, BATCH).transpose(2, 0, 1)
  gc = gc_t.reshape(HIST, LATENT_DIM, BATCH).transpose(2, 0, 1)
  return (rating, gp, gc)
